# baseline (speedup 1.0000x reference)
import functools

import jax
import jax.numpy as jnp
from jax import lax
from jax.experimental import pallas as pl
from jax.experimental.pallas import tpu as pltpu
from jax.experimental.pallas import tpu_sc as plsc

_SEED_ROWS = 8


def kernel(hist, idx, logs):
    B = hist.shape[1]
    V = logs.shape[0] - 1

    info = plsc.get_sparse_core_info()
    NC, NS = info.num_cores, info.num_subcores
    rows_per_core = B // NC
    nblk = rows_per_core // _SEED_ROWS
    blk_per_tile = -(-(nblk - 1) // NS)

    mesh = plsc.VectorSubcoreMesh(core_axis_name="c", subcore_axis_name="s")

    @functools.partial(
        pl.kernel,
        mesh=mesh,
        out_type=jax.ShapeDtypeStruct((B, V), jnp.float32),
        scratch_types=[
            pltpu.VMEM((V,), jnp.float32),
            pltpu.SemaphoreType.DMA,
        ],
    )
    def bcast(logs_hbm, out_hbm, row_v, sem):
        c = lax.axis_index("c")
        s = lax.axis_index("s")
        core_base = c * rows_per_core
        pltpu.sync_copy(logs_hbm.at[pl.ds(0, V)], row_v)

        @pl.when(s < _SEED_ROWS)
        def _():
            pltpu.sync_copy(row_v, out_hbm.at[core_base + s])

        plsc.subcore_barrier()
        seed = out_hbm.at[pl.ds(core_base, _SEED_ROWS)]
        copies = []
        for j in range(blk_per_tile):
            blk = jnp.minimum(1 + s + j * NS, nblk - 1)
            copies.append(
                pltpu.make_async_copy(
                    seed,
                    out_hbm.at[pl.ds(core_base + blk * _SEED_ROWS, _SEED_ROWS)],
                    sem,
                )
            )
        for cp in copies:
            cp.start()
        for cp in copies:
            cp.wait()

    return bcast(logs)


# speedup vs baseline: 25.0081x; 25.0081x over previous
import functools

import jax
import jax.numpy as jnp
from jax import lax
from jax.experimental import pallas as pl
from jax.experimental.pallas import tpu as pltpu
from jax.experimental.pallas import tpu_sc as plsc


def kernel(hist, idx, logs):
    B = hist.shape[1]
    V = logs.shape[0] - 1

    info = plsc.get_sparse_core_info()
    NC, NS = info.num_cores, info.num_subcores
    NW = NC * NS
    b_per_w = B // NW

    mesh = plsc.VectorSubcoreMesh(core_axis_name="c", subcore_axis_name="s")

    @functools.partial(
        pl.kernel,
        mesh=mesh,
        out_type=jax.ShapeDtypeStruct((B, V), jnp.float32),
        scratch_types=[
            pltpu.VMEM((V,), jnp.float32),
            pltpu.SemaphoreType.DMA,
        ],
    )
    def bcast(logs_hbm, out_hbm, row_v, sem):
        wid = lax.axis_index("s") * NC + lax.axis_index("c")
        pltpu.sync_copy(logs_hbm.at[pl.ds(0, V)], row_v)
        base = wid * b_per_w
        copies = [
            pltpu.make_async_copy(row_v, out_hbm.at[base + i], sem)
            for i in range(b_per_w)
        ]
        for c in copies:
            c.start()
        for c in copies:
            c.wait()

    return bcast(logs)
